# baseline (speedup 1.0000x reference)
import functools

import jax
import jax.numpy as jnp
from jax import lax
from jax.experimental import pallas as pl
from jax.experimental.pallas import tpu as pltpu
from jax.experimental.pallas import tpu_sc as plsc

N = 10000
E = 320000
D_IN = 128
H1 = 32
H2 = 64

NC = 2
NS = 16
NW = NC * NS
NP = 10112
RPS = NP // NS
C = 128
NCH = 80
NBUF = 2
EWP = NCH * C
EP = EWP * NW

W1A = 48


def _mm_a_body(xb, w1l48, w1r, o1, o2):
    xc48 = lax.dot_general(xb[...], w1l48[...], (((1,), (1,)), ((), ())),
                           preferred_element_type=jnp.float32)
    col = lax.broadcasted_iota(jnp.int32, xc48.shape, 1)
    o1[...] = xc48 + jnp.where(col == H1, 1.0, 0.0).astype(jnp.float32)
    o2[...] = lax.dot_general(xb[...], w1r[...], (((1,), (1,)), ((), ())),
                              preferred_element_type=jnp.float32)


def _mm_a(x, w1l48, w1r, bm=10000):
    grid = (N // bm,)
    return pl.pallas_call(
        _mm_a_body,
        grid=grid,
        in_specs=[
            pl.BlockSpec((bm, D_IN), lambda i: (i, 0)),
            pl.BlockSpec((W1A, D_IN), lambda i: (0, 0)),
            pl.BlockSpec((H1, D_IN), lambda i: (0, 0)),
        ],
        out_specs=[
            pl.BlockSpec((bm, W1A), lambda i: (i, 0)),
            pl.BlockSpec((bm, H1), lambda i: (i, 0)),
        ],
        out_shape=[
            jax.ShapeDtypeStruct((N, W1A), jnp.float32),
            jax.ShapeDtypeStruct((N, H1), jnp.float32),
        ],
    )(x, w1l48, w1r)


def _ep1_body(p, xr, b1l, w2l, w2r, o_hp, o_hr):
    s = p[0] + p[1]
    deg = s[:, H1:H1 + 1]
    invd = 1.0 / jnp.maximum(deg, 1.0)
    h = jnp.maximum(s[:, :H1] * invd + b1l[...] + xr[...], 0.0)
    o_hp[...] = lax.dot_general(h, w2l[...], (((1,), (1,)), ((), ())),
                                preferred_element_type=jnp.float32)
    o_hr[...] = lax.dot_general(h, w2r[...], (((1,), (1,)), ((), ())),
                                preferred_element_type=jnp.float32)


def _ep1(p, xr, b1l, w2l, w2r, bm=10000):
    grid = (N // bm,)
    return pl.pallas_call(
        _ep1_body,
        grid=grid,
        in_specs=[
            pl.BlockSpec((NC, bm, W1A), lambda i: (0, i, 0)),
            pl.BlockSpec((bm, H1), lambda i: (i, 0)),
            pl.BlockSpec((1, H1), lambda i: (0, 0)),
            pl.BlockSpec((H2, H1), lambda i: (0, 0)),
            pl.BlockSpec((H2, H1), lambda i: (0, 0)),
        ],
        out_specs=[
            pl.BlockSpec((bm, H2), lambda i: (i, 0)),
            pl.BlockSpec((bm, H2), lambda i: (i, 0)),
        ],
        out_shape=[
            jax.ShapeDtypeStruct((N, H2), jnp.float32),
            jax.ShapeDtypeStruct((N, H2), jnp.float32),
        ],
    )(p, xr, b1l, w2l, w2r)


def _ep2_body(q, p, hr, b2l, o):
    deg = (p[0] + p[1])[:, H1:H1 + 1]
    invd = 1.0 / jnp.maximum(deg, 1.0)
    o[...] = (q[0] + q[1]) * invd + b2l[...] + hr[...]


def _ep2(q, p, hr, b2l, bm=10000):
    grid = (N // bm,)
    return pl.pallas_call(
        _ep2_body,
        grid=grid,
        in_specs=[
            pl.BlockSpec((NC, bm, H2), lambda i: (0, i, 0)),
            pl.BlockSpec((NC, bm, W1A), lambda i: (0, i, 0)),
            pl.BlockSpec((bm, H2), lambda i: (i, 0)),
            pl.BlockSpec((1, H2), lambda i: (0, 0)),
        ],
        out_specs=pl.BlockSpec((bm, H2), lambda i: (i, 0)),
        out_shape=jax.ShapeDtypeStruct((N, H2), jnp.float32),
    )(q, p, hr, b2l)


@functools.lru_cache(maxsize=None)
def _make_sc_scatter(w):
    mesh = plsc.VectorSubcoreMesh(core_axis_name="c", subcore_axis_name="s")

    @functools.partial(
        pl.kernel,
        out_type=jax.ShapeDtypeStruct((NC, NP, w), jnp.float32),
        mesh=mesh,
        compiler_params=pltpu.CompilerParams(use_tc_tiling_on_sc=False),
        scratch_types=[
            pltpu.VMEM((NCH, C), jnp.int32),
            pltpu.VMEM((NCH, C), jnp.int32),
            pltpu.VMEM((NBUF, C, w), jnp.float32),
            pltpu.VMEM((RPS, w), jnp.float32),
            pltpu.VMEM_SHARED((NP, w), jnp.float32),
            [pltpu.SemaphoreType.DMA] * NBUF,
        ],
    )
    def sc_scatter(src_hbm, dst_hbm, table_hbm, zeros_hbm, out_hbm,
                   src_v, dst_v, rows_v, zv, acc, gsems):
        cid = lax.axis_index("c")
        sid = lax.axis_index("s")
        wid = sid * NC + cid
        pltpu.sync_copy(src_hbm.at[wid], src_v)
        pltpu.sync_copy(dst_hbm.at[wid], dst_v)

        def g_start(j, b):
            pltpu.make_async_copy(
                table_hbm.at[src_v.at[j]], rows_v.at[b], gsems[b]).start()

        def g_wait(b):
            pltpu.make_async_copy(
                table_hbm.at[src_v.at[0]], rows_v.at[b], gsems[b]).wait()

        for b in range(NBUF):
            g_start(b, b)
        pltpu.sync_copy(zeros_hbm.at[pl.ds(sid * RPS, RPS)], zv)
        pltpu.sync_copy(zv, acc.at[pl.ds(sid * RPS, RPS)])
        plsc.subcore_barrier()

        def group(i, carry):
            for b in range(NBUF):
                j = NBUF * i + b
                g_wait(b)
                pltpu.sync_copy(rows_v.at[b], acc.at[dst_v.at[j]], add=True)

                @pl.when(j + NBUF < NCH)
                def _():
                    g_start(j + NBUF, b)
            return carry

        lax.fori_loop(0, NCH // NBUF, group, 0)
        plsc.subcore_barrier()
        pltpu.sync_copy(acc.at[pl.ds(sid * RPS, RPS)], zv)
        pltpu.sync_copy(zv, out_hbm.at[cid, pl.ds(sid * RPS, RPS)])

    return sc_scatter


def kernel(x, edge_index, W1l, b1l, W1r, W2l, b2l, W2r):
    ppw = (EP - E) // NW
    ew0 = E // NW
    srcpad = (jnp.arange(NW * ppw, dtype=jnp.int32) % N).reshape(NW, ppw)
    dstpad = (N + jnp.arange(NW * ppw, dtype=jnp.int32) % (NP - N)).reshape(
        NW, ppw).astype(jnp.int32)
    src_r = jnp.concatenate(
        [edge_index[0].reshape(NW, ew0), srcpad], axis=1).reshape(NW, NCH, C)
    dst_r = jnp.concatenate(
        [edge_index[1].reshape(NW, ew0), dstpad], axis=1).reshape(NW, NCH, C)
    w1l48 = jnp.concatenate(
        [W1l, jnp.zeros((W1A - H1, D_IN), jnp.float32)], axis=0)
    z48 = jnp.zeros((NP, W1A), jnp.float32)
    z64 = jnp.zeros((NP, H2), jnp.float32)

    xp_aug, xr = _mm_a(x, w1l48, W1r)
    p = _make_sc_scatter(W1A)(src_r, dst_r, xp_aug, z48)
    hp_l, hr = _ep1(p, xr, b1l.reshape(1, H1), W2l, W2r)
    q = _make_sc_scatter(H2)(src_r, dst_r, hp_l, z64)
    out = _ep2(q, p, hr, b2l.reshape(1, H2))
    return out


# speedup vs baseline: 1.0119x; 1.0119x over previous
import functools

import jax
import jax.numpy as jnp
from jax import lax
from jax.experimental import pallas as pl
from jax.experimental.pallas import tpu as pltpu
from jax.experimental.pallas import tpu_sc as plsc

N = 10000
E = 320000
D_IN = 128
H1 = 32
H2 = 64

NC = 2
NS = 16
NW = NC * NS
NP = 10112
RPS = NP // NS
C = 128
NCH = 80
NBUF = 2
EWP = NCH * C
EP = EWP * NW

W1A = 48


def _mm_a_body(xb, w1l48, w1r, o1, o2):
    xc48 = lax.dot_general(xb[...], w1l48[...], (((1,), (1,)), ((), ())),
                           preferred_element_type=jnp.float32)
    col = lax.broadcasted_iota(jnp.int32, xc48.shape, 1)
    o1[...] = xc48 + jnp.where(col == H1, 1.0, 0.0).astype(jnp.float32)
    o2[...] = lax.dot_general(xb[...], w1r[...], (((1,), (1,)), ((), ())),
                              preferred_element_type=jnp.float32)


def _mm_a(x, w1l48, w1r, bm=5000):
    grid = (N // bm,)
    return pl.pallas_call(
        _mm_a_body,
        grid=grid,
        in_specs=[
            pl.BlockSpec((bm, D_IN), lambda i: (i, 0)),
            pl.BlockSpec((W1A, D_IN), lambda i: (0, 0)),
            pl.BlockSpec((H1, D_IN), lambda i: (0, 0)),
        ],
        out_specs=[
            pl.BlockSpec((bm, W1A), lambda i: (i, 0)),
            pl.BlockSpec((bm, H1), lambda i: (i, 0)),
        ],
        out_shape=[
            jax.ShapeDtypeStruct((N, W1A), jnp.float32),
            jax.ShapeDtypeStruct((N, H1), jnp.float32),
        ],
    )(x, w1l48, w1r)


def _ep1_body(p, xr, b1l, w2l, w2r, o_hp, o_hr):
    s = p[0] + p[1]
    deg = s[:, H1:H1 + 1]
    invd = 1.0 / jnp.maximum(deg, 1.0)
    h = jnp.maximum(s[:, :H1] * invd + b1l[...] + xr[...], 0.0)
    o_hp[...] = lax.dot_general(h, w2l[...], (((1,), (1,)), ((), ())),
                                preferred_element_type=jnp.float32)
    o_hr[...] = lax.dot_general(h, w2r[...], (((1,), (1,)), ((), ())),
                                preferred_element_type=jnp.float32)


def _ep1(p, xr, b1l, w2l, w2r, bm=5000):
    grid = (N // bm,)
    return pl.pallas_call(
        _ep1_body,
        grid=grid,
        in_specs=[
            pl.BlockSpec((NC, bm, W1A), lambda i: (0, i, 0)),
            pl.BlockSpec((bm, H1), lambda i: (i, 0)),
            pl.BlockSpec((1, H1), lambda i: (0, 0)),
            pl.BlockSpec((H2, H1), lambda i: (0, 0)),
            pl.BlockSpec((H2, H1), lambda i: (0, 0)),
        ],
        out_specs=[
            pl.BlockSpec((bm, H2), lambda i: (i, 0)),
            pl.BlockSpec((bm, H2), lambda i: (i, 0)),
        ],
        out_shape=[
            jax.ShapeDtypeStruct((N, H2), jnp.float32),
            jax.ShapeDtypeStruct((N, H2), jnp.float32),
        ],
    )(p, xr, b1l, w2l, w2r)


def _ep2_body(q, p, hr, b2l, o):
    deg = (p[0] + p[1])[:, H1:H1 + 1]
    invd = 1.0 / jnp.maximum(deg, 1.0)
    o[...] = (q[0] + q[1]) * invd + b2l[...] + hr[...]


def _ep2(q, p, hr, b2l, bm=5000):
    grid = (N // bm,)
    return pl.pallas_call(
        _ep2_body,
        grid=grid,
        in_specs=[
            pl.BlockSpec((NC, bm, H2), lambda i: (0, i, 0)),
            pl.BlockSpec((NC, bm, W1A), lambda i: (0, i, 0)),
            pl.BlockSpec((bm, H2), lambda i: (i, 0)),
            pl.BlockSpec((1, H2), lambda i: (0, 0)),
        ],
        out_specs=pl.BlockSpec((bm, H2), lambda i: (i, 0)),
        out_shape=jax.ShapeDtypeStruct((N, H2), jnp.float32),
    )(q, p, hr, b2l)


@functools.lru_cache(maxsize=None)
def _make_sc_scatter(w):
    mesh = plsc.VectorSubcoreMesh(core_axis_name="c", subcore_axis_name="s")

    @functools.partial(
        pl.kernel,
        out_type=jax.ShapeDtypeStruct((NC, NP, w), jnp.float32),
        mesh=mesh,
        compiler_params=pltpu.CompilerParams(use_tc_tiling_on_sc=False),
        scratch_types=[
            pltpu.VMEM((NCH, C), jnp.int32),
            pltpu.VMEM((NCH, C), jnp.int32),
            pltpu.VMEM((NBUF, C, w), jnp.float32),
            pltpu.VMEM((RPS, w), jnp.float32),
            pltpu.VMEM_SHARED((NP, w), jnp.float32),
            [pltpu.SemaphoreType.DMA] * NBUF,
        ],
    )
    def sc_scatter(src_hbm, dst_hbm, table_hbm, zeros_hbm, out_hbm,
                   src_v, dst_v, rows_v, zv, acc, gsems):
        cid = lax.axis_index("c")
        sid = lax.axis_index("s")
        wid = sid * NC + cid
        pltpu.sync_copy(src_hbm.at[wid], src_v)
        pltpu.sync_copy(dst_hbm.at[wid], dst_v)

        def g_start(j, b):
            pltpu.make_async_copy(
                table_hbm.at[src_v.at[j]], rows_v.at[b], gsems[b]).start()

        def g_wait(b):
            pltpu.make_async_copy(
                table_hbm.at[src_v.at[0]], rows_v.at[b], gsems[b]).wait()

        for b in range(NBUF):
            g_start(b, b)
        pltpu.sync_copy(zeros_hbm.at[pl.ds(sid * RPS, RPS)], zv)
        pltpu.sync_copy(zv, acc.at[pl.ds(sid * RPS, RPS)])
        plsc.subcore_barrier()

        def group(i, carry):
            for b in range(NBUF):
                j = NBUF * i + b
                g_wait(b)
                pltpu.sync_copy(rows_v.at[b], acc.at[dst_v.at[j]], add=True)

                @pl.when(j + NBUF < NCH)
                def _():
                    g_start(j + NBUF, b)
            return carry

        lax.fori_loop(0, NCH // NBUF, group, 0)
        plsc.subcore_barrier()
        pltpu.sync_copy(acc.at[pl.ds(sid * RPS, RPS)], zv)
        pltpu.sync_copy(zv, out_hbm.at[cid, pl.ds(sid * RPS, RPS)])

    return sc_scatter


def kernel(x, edge_index, W1l, b1l, W1r, W2l, b2l, W2r):
    ppw = (EP - E) // NW
    ew0 = E // NW
    srcpad = (jnp.arange(NW * ppw, dtype=jnp.int32) % N).reshape(NW, ppw)
    dstpad = (N + jnp.arange(NW * ppw, dtype=jnp.int32) % (NP - N)).reshape(
        NW, ppw).astype(jnp.int32)
    src_r = jnp.concatenate(
        [edge_index[0].reshape(NW, ew0), srcpad], axis=1).reshape(NW, NCH, C)
    dst_r = jnp.concatenate(
        [edge_index[1].reshape(NW, ew0), dstpad], axis=1).reshape(NW, NCH, C)
    w1l48 = jnp.concatenate(
        [W1l, jnp.zeros((W1A - H1, D_IN), jnp.float32)], axis=0)
    z48 = jnp.zeros((NP, W1A), jnp.float32)
    z64 = jnp.zeros((NP, H2), jnp.float32)

    xp_aug, xr = _mm_a(x, w1l48, W1r)
    p = _make_sc_scatter(W1A)(src_r, dst_r, xp_aug, z48)
    hp_l, hr = _ep1(p, xr, b1l.reshape(1, H1), W2l, W2r)
    q = _make_sc_scatter(H2)(src_r, dst_r, hp_l, z64)
    out = _ep2(q, p, hr, b2l.reshape(1, H2))
    return out
